# baseline (speedup 1.0000x reference)
import functools

import jax
import jax.numpy as jnp
import numpy as np
from jax import lax
from jax.experimental import pallas as pl
from jax.experimental.pallas import tpu as pltpu
from jax.experimental.pallas import tpu_sc as plsc

_N = 10000
_E = 320000
_FIN = 128
_HID = 32
_NCH = 8
_CST = 1e-05

_HALF = (_HID * _NCH + _HID) // 2
_HPAD = 160
_CHUNK = 128
_NSUB = 16
_C = 64
_CPS = 16
_SLAB = _C * _CPS
_EPAD = (_NSUB * _SLAB) * -(-_E // (_NSUB * _SLAB))
_EPT = _EPAD // _NSUB
_NSLAB = _EPT // _SLAB
_EPW = _EPAD // (2 * _NSUB)
_NPAD = 640 * _NSUB
_RPT = _NPAD // _NSUB

_BN = 1000
_GRID = _N // _BN


def _sc_mesh():
    return plsc.VectorSubcoreMesh(core_axis_name="c", subcore_axis_name="s",
                                  num_cores=2, num_subcores=_NSUB)


@functools.cache
def _get_deg_kernel():
    return functools.partial(
        pl.kernel,
        out_type=jax.ShapeDtypeStruct((2, _NPAD), jnp.float32),
        mesh=_sc_mesh(),
        scratch_types=[
            pltpu.VMEM((_EPW,), jnp.int32),
            pltpu.VMEM((_CHUNK,), jnp.int32),
            pltpu.VMEM((_CHUNK,), jnp.float32),
            pltpu.VMEM((_RPT,), jnp.float32),
            pltpu.VMEM_SHARED((_NPAD,), jnp.float32),
        ],
        compiler_params=pltpu.CompilerParams(use_tc_tiling_on_sc=False),
    )(_deg_body)


def _deg_body(colp, out, slab, cidx, ones, zbuf, accd):
    c = lax.axis_index("c")
    s = lax.axis_index("s")
    w = s * 2 + c
    pltpu.sync_copy(colp.at[pl.ds(w * _EPW, _EPW)], slab)
    for j in range(_CHUNK // 16):
        ones[pl.ds(j * 16, 16)] = jnp.full((16,), 1.0, jnp.float32)
    for i in range(_RPT // 16):
        zbuf[pl.ds(i * 16, 16)] = jnp.zeros((16,), jnp.float32)
    pltpu.sync_copy(zbuf, accd.at[pl.ds(s * _RPT, _RPT)])
    plsc.subcore_barrier()

    def chunk(k, carry):
        e0 = k * _CHUNK
        for j in range(_CHUNK // 16):
            cidx[pl.ds(j * 16, 16)] = slab[pl.ds(e0 + j * 16, 16)]
        pltpu.sync_copy(ones, accd.at[cidx], add=True)
        return carry

    lax.fori_loop(0, _EPW // _CHUNK, chunk, 0)
    plsc.subcore_barrier()
    pltpu.sync_copy(accd.at[pl.ds(s * _RPT, _RPT)], out.at[c, pl.ds(s * _RPT, _RPT)])


@functools.cache
def _get_hop_kernel():
    return functools.partial(
        pl.kernel,
        out_type=jax.ShapeDtypeStruct((2, _NPAD, _HALF), jnp.float32),
        mesh=_sc_mesh(),
        scratch_types=[
            pltpu.VMEM((_SLAB,), jnp.int32),
            pltpu.VMEM((_SLAB,), jnp.int32),
            pltpu.VMEM((_C,), jnp.int32),
            pltpu.VMEM((_C,), jnp.int32),
            pltpu.VMEM((_C,), jnp.int32),
            pltpu.VMEM((_C,), jnp.int32),
            pltpu.VMEM((_C,), jnp.int32),
            pltpu.VMEM((_C, _HPAD), jnp.bfloat16),
            pltpu.VMEM((_C, _HPAD), jnp.bfloat16),
            pltpu.VMEM((_C, _HPAD), jnp.bfloat16),
            pltpu.VMEM((_C, _HALF), jnp.float32),
            pltpu.VMEM((_C, _HALF), jnp.float32),
            pltpu.VMEM_SHARED((_NPAD, _HALF), jnp.float32),
            pltpu.SemaphoreType.DMA,
            pltpu.SemaphoreType.DMA,
            pltpu.SemaphoreType.DMA,
            pltpu.SemaphoreType.DMA,
            pltpu.SemaphoreType.DMA,
        ],
        compiler_params=pltpu.CompilerParams(use_tc_tiling_on_sc=False,
                                             needs_layout_passes=False),
    )(_hop_body)


def _hop_body(ztab, rowp, colp, out, erow, ecol, g0, g1, g2, x0, x1,
              b0, b1, b2, f0, f1, acc, s0, s1, s2, t0, t1):
    c = lax.axis_index("c")
    s = lax.axis_index("s")
    gx, cx = (g0, g1, g2), (x0, x1)
    bb, fb = (b0, b1, b2), (f0, f1)
    gsm, ssm = (s0, s1, s2), (t0, t1)

    def unpack_rows(bsrc, fdst):
        def row_fn(r, carry):
            for g in range(_HPAD // 32):
                pair = bsrc[r, pl.ds(32 * g, 32)]
                lo, hi = plsc.unpack(pair, format=plsc.PackFormat.INTERLEAVED)
                fdst[r, pl.ds(32 * g, 16)] = lo
                if 32 * g + 16 < _HALF:
                    fdst[r, pl.ds(32 * g + 16, 16)] = hi
            return carry

        lax.fori_loop(0, _C, row_fn, 0)

    nb = s * _RPT
    for q in range(_RPT // _C):
        base = nb + q * _C
        for j in range(_C // 16):
            idxv = base + j * 16 + lax.iota(jnp.int32, 16)
            idxv = jnp.minimum(idxv, _N - 1)
            g0[pl.ds(j * 16, 16)] = idxv * 2 + c
        pltpu.async_copy(ztab.at[g0], b0, s0).wait()
        unpack_rows(b0, f0)
        pltpu.sync_copy(f0, acc.at[pl.ds(base, _C)])
    plsc.subcore_barrier()

    ebase = s * _EPT

    def build_g(k, g):
        k0 = k * _C
        for j in range(_C // 16):
            rv = erow[pl.ds(k0 + j * 16, 16)]
            g[pl.ds(j * 16, 16)] = rv * 2 + c

    def build_c(k, x):
        k0 = k * _C
        for j in range(_C // 16):
            x[pl.ds(j * 16, 16)] = ecol[pl.ds(k0 + j * 16, 16)]

    def slab_fn(si, carry):
        e0 = ebase + si * _SLAB
        pltpu.sync_copy(rowp.at[pl.ds(e0, _SLAB)], erow)
        pltpu.sync_copy(colp.at[pl.ds(e0, _SLAB)], ecol)
        dg = [None, None, None]
        ds = [None, None]

        def consume(k):
            q3, q2 = k % 3, k % 2
            dg[q3].wait()
            unpack_rows(bb[q3], fb[q2])
            build_c(k, cx[q2])
            ds[q2] = pltpu.async_copy(fb[q2], acc.at[cx[q2]], ssm[q2],
                                      add=True)

        for k in range(_CPS):
            p3, p2 = k % 3, k % 2
            if k >= 2:
                ds[p2].wait()
            build_g(k, gx[p3])
            dg[p3] = pltpu.async_copy(ztab.at[gx[p3]], bb[p3], gsm[p3])
            if k >= 1:
                consume(k - 1)
        ds[(_CPS - 2) % 2].wait()
        consume(_CPS - 1)
        ds[(_CPS - 1) % 2].wait()
        return carry

    lax.fori_loop(0, _NSLAB, slab_fn, 0)
    plsc.subcore_barrier()

    def wb(m, carry):
        base = s * _RPT + m * 16

        @pl.when(base < _N)
        def _():
            pltpu.sync_copy(acc.at[pl.ds(base, 16)], out.at[c, pl.ds(base, 16)])

        return carry

    lax.fori_loop(0, _RPT // 16, wb, 0)


def _elu1(z):
    return 1.0 + jnp.where(z > 0, z, jnp.exp(z) - 1.0)


def _prep_body(x, wi, bi, wq, bq, wk, bk, wv, bv, rr, tt, d0, d1, tmp, pm,
               z_o, q_o, h0_o, dis_o):
    h = jnp.maximum(jnp.dot(x[...], wi[...], preferred_element_type=jnp.float32)
                    + bi[...], 0.0)
    q = _elu1(jnp.dot(h, wq[...], preferred_element_type=jnp.float32) + bq[...])
    kf = _elu1(jnp.dot(h, wk[...], preferred_element_type=jnp.float32) + bk[...])
    v = jnp.dot(h, wv[...], preferred_element_type=jnp.float32) + bv[...]
    deg = d0[...] + d1[...] + 1.0
    dis = lax.rsqrt(deg)
    zm = (jnp.dot(kf, rr[...], preferred_element_type=jnp.float32)
          * jnp.dot(v, tt[...], preferred_element_type=jnp.float32))
    zfull = jnp.concatenate([zm, kf], axis=1) * dis
    z_o[...] = jnp.dot(zfull, pm[...],
                       preferred_element_type=jnp.float32).astype(jnp.bfloat16)
    q_o[...] = q
    h0_o[...] = v * tmp[0, 0]
    dis_o[...] = dis


def _prep_call(x, wi, bi, wq, bq, wk, bk, wv, bv, rr, tt, d0, d1, tmp, pm):
    full = lambda shape: pl.BlockSpec(shape, lambda i: (0, 0))
    rowb = lambda w: pl.BlockSpec((_BN, w), lambda i: (i, 0))
    return pl.pallas_call(
        _prep_body,
        grid=(_GRID,),
        in_specs=[
            rowb(_FIN), full((_FIN, _HID)), full((1, _HID)),
            full((_HID, _HID)), full((1, _HID)),
            full((_HID, _HID)), full((1, _HID)),
            full((_HID, _NCH)), full((1, _NCH)),
            full((_HID, _HID * _NCH)), full((_NCH, _HID * _NCH)),
            rowb(1), rowb(1), full((1, 3)), full((2 * _HALF, 2 * _HPAD)),
        ],
        out_specs=[rowb(2 * _HPAD), rowb(_HID), rowb(_NCH), rowb(1)],
        out_shape=[
            jax.ShapeDtypeStruct((_N, 2 * _HPAD), jnp.bfloat16),
            jax.ShapeDtypeStruct((_N, _HID), jnp.float32),
            jax.ShapeDtypeStruct((_N, _NCH), jnp.float32),
            jax.ShapeDtypeStruct((_N, 1), jnp.float32),
        ],
    )(x, wi, bi, wq, bq, wk, bk, wv, bv, rr, tt, d0, d1, tmp, pm)


def _post_body(ya, yb, q, dis, hid, rr, ttt, tmp, pm, ho_o, z_o):
    y = jnp.concatenate([ya[0], yb[0]], axis=1)
    d = dis[...]
    m = y[:, : _HID * _NCH] * d
    kf = y[:, _HID * _NCH:] * d
    qb = q[...]
    hh = jnp.dot(jnp.dot(qb, rr[...], preferred_element_type=jnp.float32) * m,
                 ttt[...], preferred_element_type=jnp.float32)
    cd = jnp.sum(qb * kf, axis=1, keepdims=True) + _CST
    ho_o[...] = hid[...] + tmp[0, 0] * (hh / cd)
    z_o[...] = jnp.dot(y * (d * d), pm[...],
                       preferred_element_type=jnp.float32).astype(jnp.bfloat16)


def _post_call(y, q, dis, hid, rr, ttt, tmp, pm):
    full = lambda shape: pl.BlockSpec(shape, lambda i: (0, 0))
    rowb = lambda w: pl.BlockSpec((_BN, w), lambda i: (i, 0))
    ya_spec = pl.BlockSpec((1, _BN, _HALF), lambda i: (0, i, 0))
    yb_spec = pl.BlockSpec((1, _BN, _HALF), lambda i: (1, i, 0))
    return pl.pallas_call(
        _post_body,
        grid=(_GRID,),
        in_specs=[
            ya_spec, yb_spec, rowb(_HID), rowb(1), rowb(_NCH),
            full((_HID, _HID * _NCH)), full((_HID * _NCH, _NCH)), full((1, 1)),
            full((2 * _HALF, 2 * _HPAD)),
        ],
        out_specs=[rowb(_NCH), rowb(2 * _HPAD)],
        out_shape=[
            jax.ShapeDtypeStruct((_N, _NCH), jnp.float32),
            jax.ShapeDtypeStruct((_N, 2 * _HPAD), jnp.bfloat16),
        ],
    )(y, y, q, dis, hid, rr, ttt, tmp, pm)


_R_EXPAND = np.kron(np.eye(_HID), np.ones((1, _NCH))).astype(np.float32)
_T_EXPAND = np.kron(np.ones((1, _HID)), np.eye(_NCH)).astype(np.float32)


def _make_pmat():
    p = np.zeros((2 * _HALF, 2 * _HPAD), np.float32)
    for c in range(2):
        for g in range(_HPAD // 32):
            for i in range(16):
                s_lo = 32 * g + i
                if s_lo < _HALF:
                    p[c * _HALF + s_lo, c * _HPAD + 32 * g + 2 * i] = 1.0
                s_hi = 32 * g + 16 + i
                if s_hi < _HALF:
                    p[c * _HALF + s_hi, c * _HPAD + 32 * g + 2 * i + 1] = 1.0
    return p


_PMAT = _make_pmat()


def kernel(x, edge_index, W_in, b_in, WQ, bQ, WK, bK, WV, bV, temp):
    row = edge_index[0]
    col = edge_index[1]
    npad = _EPAD - _E
    rowp = jnp.concatenate([row, jnp.zeros((npad,), jnp.int32)])
    colp = jnp.concatenate([col, jnp.full((npad,), _N, jnp.int32)])

    degp = _get_deg_kernel()(colp)
    d0 = degp[0, :_N, None]
    d1 = degp[1, :_N, None]

    tmp = temp.reshape(1, 3)
    z0, q, hid0, dis = _prep_call(
        x, W_in.T, b_in.reshape(1, _HID), WQ.T, bQ.reshape(1, _HID),
        WK.T, bK.reshape(1, _HID), WV.T, bV.reshape(1, _NCH),
        _R_EXPAND, _T_EXPAND, d0, d1, tmp, _PMAT)

    ttt = np.ascontiguousarray(_T_EXPAND.T)
    t_steps = temp[1:].reshape(2, 1, 1)

    def body(carry, t_h):
        hid, z = carry
        y = _get_hop_kernel()(z.reshape(2 * _N, _HPAD), rowp, colp)
        hid_n, z_n = _post_call(y, q, dis, hid, _R_EXPAND, ttt, t_h, _PMAT)
        return (hid_n, z_n), None

    (hid_f, _), _ = lax.scan(body, (hid0, z0), t_steps)
    return hid_f


# speedup vs baseline: 1.2901x; 1.2901x over previous
import functools

import jax
import jax.numpy as jnp
import numpy as np
from jax import lax
from jax.experimental import pallas as pl
from jax.experimental.pallas import tpu as pltpu
from jax.experimental.pallas import tpu_sc as plsc

_N = 10000
_E = 320000
_FIN = 128
_HID = 32
_NCH = 8
_CST = 1e-05

_HALF = (_HID * _NCH + _HID) // 2
_CHUNK = 128
_NSUB = 16
_C = 80
_CPS = 16
_SLAB = _C * _CPS
_EPAD = (_NSUB * _SLAB) * -(-_E // (_NSUB * _SLAB))
_EPT = _EPAD // _NSUB
_NSLAB = _EPT // _SLAB
_EPW = _EPAD // (2 * _NSUB)
_NPAD = 640 * _NSUB
_RPT = _NPAD // _NSUB

_BN = 1000
_GRID = _N // _BN


def _sc_mesh():
    return plsc.VectorSubcoreMesh(core_axis_name="c", subcore_axis_name="s",
                                  num_cores=2, num_subcores=_NSUB)


@functools.cache
def _get_deg_kernel():
    return functools.partial(
        pl.kernel,
        out_type=jax.ShapeDtypeStruct((2, _NPAD), jnp.float32),
        mesh=_sc_mesh(),
        scratch_types=[
            pltpu.VMEM((_EPW,), jnp.int32),
            pltpu.VMEM((_CHUNK,), jnp.int32),
            pltpu.VMEM((_CHUNK,), jnp.float32),
            pltpu.VMEM((_RPT,), jnp.float32),
            pltpu.VMEM_SHARED((_NPAD,), jnp.float32),
        ],
        compiler_params=pltpu.CompilerParams(use_tc_tiling_on_sc=False),
    )(_deg_body)


def _deg_body(colp, out, slab, cidx, ones, zbuf, accd):
    c = lax.axis_index("c")
    s = lax.axis_index("s")
    w = s * 2 + c
    pltpu.sync_copy(colp.at[pl.ds(w * _EPW, _EPW)], slab)
    for j in range(_CHUNK // 16):
        ones[pl.ds(j * 16, 16)] = jnp.full((16,), 1.0, jnp.float32)
    for i in range(_RPT // 16):
        zbuf[pl.ds(i * 16, 16)] = jnp.zeros((16,), jnp.float32)
    pltpu.sync_copy(zbuf, accd.at[pl.ds(s * _RPT, _RPT)])
    plsc.subcore_barrier()

    def chunk(k, carry):
        e0 = k * _CHUNK
        for j in range(_CHUNK // 16):
            cidx[pl.ds(j * 16, 16)] = slab[pl.ds(e0 + j * 16, 16)]
        pltpu.sync_copy(ones, accd.at[cidx], add=True)
        return carry

    lax.fori_loop(0, _EPW // _CHUNK, chunk, 0)
    plsc.subcore_barrier()
    pltpu.sync_copy(accd.at[pl.ds(s * _RPT, _RPT)], out.at[c, pl.ds(s * _RPT, _RPT)])


@functools.cache
def _get_hop_kernel():
    return functools.partial(
        pl.kernel,
        out_type=jax.ShapeDtypeStruct((2, _NPAD, _HALF), jnp.float32),
        mesh=_sc_mesh(),
        scratch_types=[
            pltpu.VMEM((_SLAB,), jnp.int32),
            pltpu.VMEM((_SLAB,), jnp.int32),
            pltpu.VMEM((_C,), jnp.int32),
            pltpu.VMEM((_C,), jnp.int32),
            pltpu.VMEM((_C,), jnp.int32),
            pltpu.VMEM((_C,), jnp.int32),
            pltpu.VMEM((_C,), jnp.int32),
            pltpu.VMEM((_C,), jnp.int32),
            pltpu.VMEM((_C, _HALF), jnp.float32),
            pltpu.VMEM((_C, _HALF), jnp.float32),
            pltpu.VMEM((_C, _HALF), jnp.float32),
            pltpu.VMEM_SHARED((_NPAD, _HALF), jnp.float32),
            pltpu.SemaphoreType.DMA,
            pltpu.SemaphoreType.DMA,
            pltpu.SemaphoreType.DMA,
            pltpu.SemaphoreType.DMA,
            pltpu.SemaphoreType.DMA,
            pltpu.SemaphoreType.DMA,
        ],
        compiler_params=pltpu.CompilerParams(use_tc_tiling_on_sc=False),
    )(_hop_body)


def _hop_body(ztab, rowp, colp, out, erow, ecol, g0, g1, g2, x0, x1, x2,
              b0, b1, b2, acc, s0, s1, s2, t0, t1, t2):
    c = lax.axis_index("c")
    s = lax.axis_index("s")
    gx, cx, bf = (g0, g1, g2), (x0, x1, x2), (b0, b1, b2)
    gsm, ssm = (s0, s1, s2), (t0, t1, t2)

    nb = s * _RPT
    nq = _RPT // _C

    def init_idx(q, g):
        base = nb + q * _C
        for j in range(_C // 16):
            idxv = base + j * 16 + lax.iota(jnp.int32, 16)
            idxv = jnp.minimum(idxv, _N - 1)
            g[pl.ds(j * 16, 16)] = idxv * 2 + c

    di = [None, None, None]
    for q in range(min(3, nq)):
        init_idx(q, gx[q])
        di[q] = pltpu.async_copy(ztab.at[gx[q]], bf[q], gsm[q])
    for q in range(nq):
        p = q % 3
        di[p].wait()
        pltpu.sync_copy(bf[p], acc.at[pl.ds(nb + q * _C, _C)])
        if q + 3 < nq:
            init_idx(q + 3, gx[p])
            di[p] = pltpu.async_copy(ztab.at[gx[p]], bf[p], gsm[p])
    plsc.subcore_barrier()

    ebase = s * _EPT

    def build(k, g, x):
        k0 = k * _C
        for j in range(_C // 16):
            rv = erow[pl.ds(k0 + j * 16, 16)]
            g[pl.ds(j * 16, 16)] = rv * 2 + c
            x[pl.ds(j * 16, 16)] = ecol[pl.ds(k0 + j * 16, 16)]

    def slab_fn(si, carry):
        e0 = ebase + si * _SLAB
        pltpu.sync_copy(rowp.at[pl.ds(e0, _SLAB)], erow)
        pltpu.sync_copy(colp.at[pl.ds(e0, _SLAB)], ecol)
        dg = [None, None, None]
        ds = [None, None, None]
        for k in range(_CPS):
            p = k % 3
            if k >= 3:
                ds[p].wait()
            build(k, gx[p], cx[p])
            dg[p] = pltpu.async_copy(ztab.at[gx[p]], bf[p], gsm[p])
            if k >= 1:
                q = (k - 1) % 3
                dg[q].wait()
                ds[q] = pltpu.async_copy(bf[q], acc.at[cx[q]], ssm[q],
                                         add=True)
        q = (_CPS - 1) % 3
        dg[q].wait()
        ds[q] = pltpu.async_copy(bf[q], acc.at[cx[q]], ssm[q], add=True)
        for k in range(_CPS - 3, _CPS):
            ds[k % 3].wait()
        return carry

    lax.fori_loop(0, _NSLAB, slab_fn, 0)
    plsc.subcore_barrier()

    last = _N - (_NSUB - 1) * _RPT

    @pl.when(s < _NSUB - 1)
    def _():
        pltpu.sync_copy(acc.at[pl.ds(s * _RPT, _RPT)],
                        out.at[c, pl.ds(s * _RPT, _RPT)])

    @pl.when(s == _NSUB - 1)
    def _():
        pltpu.sync_copy(acc.at[pl.ds((_NSUB - 1) * _RPT, last)],
                        out.at[c, pl.ds((_NSUB - 1) * _RPT, last)])


def _elu1(z):
    return 1.0 + jnp.where(z > 0, z, jnp.exp(z) - 1.0)


def _prep_body(x, wi, bi, wq, bq, wk, bk, wv, bv, rr, tt, d0, d1, tmp,
               z_o, q_o, h0_o, dis_o):
    h = jnp.maximum(jnp.dot(x[...], wi[...], preferred_element_type=jnp.float32)
                    + bi[...], 0.0)
    q = _elu1(jnp.dot(h, wq[...], preferred_element_type=jnp.float32) + bq[...])
    kf = _elu1(jnp.dot(h, wk[...], preferred_element_type=jnp.float32) + bk[...])
    v = jnp.dot(h, wv[...], preferred_element_type=jnp.float32) + bv[...]
    deg = d0[...] + d1[...] + 1.0
    dis = lax.rsqrt(deg)
    zm = (jnp.dot(kf, rr[...], preferred_element_type=jnp.float32)
          * jnp.dot(v, tt[...], preferred_element_type=jnp.float32))
    z_o[...] = jnp.concatenate([zm, kf], axis=1) * dis
    q_o[...] = q
    h0_o[...] = v * tmp[0, 0]
    dis_o[...] = dis


def _prep_call(x, wi, bi, wq, bq, wk, bk, wv, bv, rr, tt, d0, d1, tmp):
    full = lambda shape: pl.BlockSpec(shape, lambda i: (0, 0))
    rowb = lambda w: pl.BlockSpec((_BN, w), lambda i: (i, 0))
    return pl.pallas_call(
        _prep_body,
        grid=(_GRID,),
        in_specs=[
            rowb(_FIN), full((_FIN, _HID)), full((1, _HID)),
            full((_HID, _HID)), full((1, _HID)),
            full((_HID, _HID)), full((1, _HID)),
            full((_HID, _NCH)), full((1, _NCH)),
            full((_HID, _HID * _NCH)), full((_NCH, _HID * _NCH)),
            rowb(1), rowb(1), full((1, 3)),
        ],
        out_specs=[rowb(2 * _HALF), rowb(_HID), rowb(_NCH), rowb(1)],
        out_shape=[
            jax.ShapeDtypeStruct((_N, 2 * _HALF), jnp.float32),
            jax.ShapeDtypeStruct((_N, _HID), jnp.float32),
            jax.ShapeDtypeStruct((_N, _NCH), jnp.float32),
            jax.ShapeDtypeStruct((_N, 1), jnp.float32),
        ],
    )(x, wi, bi, wq, bq, wk, bk, wv, bv, rr, tt, d0, d1, tmp)


def _post_body(ya, yb, q, dis, hid, rr, ttt, tmp, ho_o, z_o):
    y = jnp.concatenate([ya[0], yb[0]], axis=1)
    d = dis[...]
    m = y[:, : _HID * _NCH] * d
    kf = y[:, _HID * _NCH:] * d
    qb = q[...]
    hh = jnp.dot(jnp.dot(qb, rr[...], preferred_element_type=jnp.float32) * m,
                 ttt[...], preferred_element_type=jnp.float32)
    cd = jnp.sum(qb * kf, axis=1, keepdims=True) + _CST
    ho_o[...] = hid[...] + tmp[0, 0] * (hh / cd)
    z_o[...] = y * (d * d)


def _post_call(y, q, dis, hid, rr, ttt, tmp):
    full = lambda shape: pl.BlockSpec(shape, lambda i: (0, 0))
    rowb = lambda w: pl.BlockSpec((_BN, w), lambda i: (i, 0))
    ya_spec = pl.BlockSpec((1, _BN, _HALF), lambda i: (0, i, 0))
    yb_spec = pl.BlockSpec((1, _BN, _HALF), lambda i: (1, i, 0))
    return pl.pallas_call(
        _post_body,
        grid=(_GRID,),
        in_specs=[
            ya_spec, yb_spec, rowb(_HID), rowb(1), rowb(_NCH),
            full((_HID, _HID * _NCH)), full((_HID * _NCH, _NCH)), full((1, 1)),
        ],
        out_specs=[rowb(_NCH), rowb(2 * _HALF)],
        out_shape=[
            jax.ShapeDtypeStruct((_N, _NCH), jnp.float32),
            jax.ShapeDtypeStruct((_N, 2 * _HALF), jnp.float32),
        ],
    )(y, y, q, dis, hid, rr, ttt, tmp)


_R_EXPAND = np.kron(np.eye(_HID), np.ones((1, _NCH))).astype(np.float32)
_T_EXPAND = np.kron(np.ones((1, _HID)), np.eye(_NCH)).astype(np.float32)


def kernel(x, edge_index, W_in, b_in, WQ, bQ, WK, bK, WV, bV, temp):
    row = edge_index[0]
    col = edge_index[1]
    npad = _EPAD - _E
    rowp = jnp.concatenate([row, jnp.zeros((npad,), jnp.int32)])
    colp = jnp.concatenate([col, jnp.full((npad,), _N, jnp.int32)])

    degp = _get_deg_kernel()(colp)
    d0 = degp[0, :_N, None]
    d1 = degp[1, :_N, None]

    tmp = temp.reshape(1, 3)
    z0, q, hid0, dis = _prep_call(
        x, W_in.T, b_in.reshape(1, _HID), WQ.T, bQ.reshape(1, _HID),
        WK.T, bK.reshape(1, _HID), WV.T, bV.reshape(1, _NCH),
        _R_EXPAND, _T_EXPAND, d0, d1, tmp)

    ttt = np.ascontiguousarray(_T_EXPAND.T)
    t_steps = temp[1:].reshape(2, 1, 1)

    def body(carry, t_h):
        hid, z = carry
        y = _get_hop_kernel()(z.reshape(2 * _N, _HALF), rowp, colp)
        hid_n, z_n = _post_call(y, q, dis, hid, _R_EXPAND, ttt, t_h)
        return (hid_n, z_n), None

    (hid_f, _), _ = lax.scan(body, (hid0, z0), t_steps)
    return hid_f


# speedup vs baseline: 2.0920x; 1.6217x over previous
import functools

import jax
import jax.numpy as jnp
import numpy as np
from jax import lax
from jax.experimental import pallas as pl
from jax.experimental.pallas import tpu as pltpu
from jax.experimental.pallas import tpu_sc as plsc

_N = 10000
_E = 320000
_FIN = 128
_HID = 32
_NCH = 8
_CST = 1e-05

_HALF = (_HID * _NCH + _HID) // 2
_CHUNK = 80
_NSUB = 16
_C = 80
_CPS = 16
_SLAB = _C * _CPS
_NCHK = 3 * -(-_E // (_NSUB * _C * 3))
_EPT = _NCHK * _C
_EPAD = _NSUB * _EPT
_EARR = _EPAD + _SLAB
_EPW = _EPAD // (2 * _NSUB)
_NPAD = 640 * _NSUB
_RPT = _NPAD // _NSUB

_BN = 1000
_GRID = _N // _BN


def _sc_mesh():
    return plsc.VectorSubcoreMesh(core_axis_name="c", subcore_axis_name="s",
                                  num_cores=2, num_subcores=_NSUB)


@functools.cache
def _get_deg_kernel():
    return functools.partial(
        pl.kernel,
        out_type=jax.ShapeDtypeStruct((2, _NPAD), jnp.float32),
        mesh=_sc_mesh(),
        scratch_types=[
            pltpu.VMEM((_EPW,), jnp.int32),
            pltpu.VMEM((_CHUNK,), jnp.int32),
            pltpu.VMEM((_CHUNK,), jnp.float32),
            pltpu.VMEM((_RPT,), jnp.float32),
            pltpu.VMEM_SHARED((_NPAD,), jnp.float32),
        ],
        compiler_params=pltpu.CompilerParams(use_tc_tiling_on_sc=False),
    )(_deg_body)


def _deg_body(colp, out, slab, cidx, ones, zbuf, accd):
    c = lax.axis_index("c")
    s = lax.axis_index("s")
    w = s * 2 + c
    pltpu.sync_copy(colp.at[pl.ds(w * _EPW, _EPW)], slab)
    for j in range(_CHUNK // 16):
        ones[pl.ds(j * 16, 16)] = jnp.full((16,), 1.0, jnp.float32)
    for i in range(_RPT // 16):
        zbuf[pl.ds(i * 16, 16)] = jnp.zeros((16,), jnp.float32)
    pltpu.sync_copy(zbuf, accd.at[pl.ds(s * _RPT, _RPT)])
    plsc.subcore_barrier()

    def chunk(k, carry):
        e0 = k * _CHUNK
        for j in range(_CHUNK // 16):
            cidx[pl.ds(j * 16, 16)] = slab[pl.ds(e0 + j * 16, 16)]
        pltpu.sync_copy(ones, accd.at[cidx], add=True)
        return carry

    lax.fori_loop(0, _EPW // _CHUNK, chunk, 0)
    plsc.subcore_barrier()
    pltpu.sync_copy(accd.at[pl.ds(s * _RPT, _RPT)], out.at[c, pl.ds(s * _RPT, _RPT)])


@functools.cache
def _get_hop_kernel():
    return functools.partial(
        pl.kernel,
        out_type=jax.ShapeDtypeStruct((2, _NPAD, _HALF), jnp.float32),
        mesh=_sc_mesh(),
        scratch_types=[
            pltpu.VMEM((_SLAB,), jnp.int32),
            pltpu.VMEM((_SLAB,), jnp.int32),
            pltpu.VMEM((_C,), jnp.int32),
            pltpu.VMEM((_C,), jnp.int32),
            pltpu.VMEM((_C,), jnp.int32),
            pltpu.VMEM((_C,), jnp.int32),
            pltpu.VMEM((_C,), jnp.int32),
            pltpu.VMEM((_C,), jnp.int32),
            pltpu.VMEM((_C, _HALF), jnp.float32),
            pltpu.VMEM((_C, _HALF), jnp.float32),
            pltpu.VMEM((_C, _HALF), jnp.float32),
            pltpu.VMEM_SHARED((_NPAD, _HALF), jnp.float32),
            pltpu.SemaphoreType.DMA,
            pltpu.SemaphoreType.DMA,
            pltpu.SemaphoreType.DMA,
            pltpu.SemaphoreType.DMA,
            pltpu.SemaphoreType.DMA,
            pltpu.SemaphoreType.DMA,
        ],
        compiler_params=pltpu.CompilerParams(use_tc_tiling_on_sc=False),
    )(_hop_body)


def _hop_body(ztab, rowp, colp, out, erow, ecol, g0, g1, g2, x0, x1, x2,
              b0, b1, b2, acc, s0, s1, s2, t0, t1, t2):
    c = lax.axis_index("c")
    s = lax.axis_index("s")
    gx, cx, bf = (g0, g1, g2), (x0, x1, x2), (b0, b1, b2)
    gsm, ssm = (s0, s1, s2), (t0, t1, t2)

    nb = s * _RPT
    nq = _RPT // _C

    def init_idx(q, g):
        base = nb + q * _C
        for j in range(_C // 16):
            idxv = base + j * 16 + lax.iota(jnp.int32, 16)
            idxv = jnp.minimum(idxv, _N - 1)
            g[pl.ds(j * 16, 16)] = idxv * 2 + c

    di = [None, None, None]
    for q in range(min(3, nq)):
        init_idx(q, gx[q])
        di[q] = pltpu.async_copy(ztab.at[gx[q]], bf[q], gsm[q])
    for q in range(nq):
        p = q % 3
        di[p].wait()
        pltpu.sync_copy(bf[p], acc.at[pl.ds(nb + q * _C, _C)])
        if q + 3 < nq:
            init_idx(q + 3, gx[p])
            di[p] = pltpu.async_copy(ztab.at[gx[p]], bf[p], gsm[p])
    plsc.subcore_barrier()

    ebase = s * _EPT

    def build(kloc, g, x):
        k0 = kloc * _C
        for j in range(_C // 16):
            rv = erow[pl.ds(k0 + j * 16, 16)]
            g[pl.ds(j * 16, 16)] = rv * 2 + c
            x[pl.ds(j * 16, 16)] = ecol[pl.ds(k0 + j * 16, 16)]

    def wait_gather(p):
        pltpu.make_async_copy(ztab.at[gx[p]], bf[p], gsm[p]).wait()

    def wait_scatter(p):
        pltpu.make_async_copy(bf[p], acc.at[cx[p]], ssm[p]).wait()

    def group_fn(gi, carry):
        for u in range(3):
            k = gi * 3 + u

            @pl.when(k % _CPS == 0)
            def _():
                e0 = ebase + (k // _CPS) * _SLAB
                pltpu.sync_copy(rowp.at[pl.ds(e0, _SLAB)], erow)
                pltpu.sync_copy(colp.at[pl.ds(e0, _SLAB)], ecol)

            @pl.when(gi >= 1)
            def _():
                wait_scatter(u)
            build(k % _CPS, gx[u], cx[u])
            pltpu.async_copy(ztab.at[gx[u]], bf[u], gsm[u])
            q = (u - 1) % 3
            if u == 0:
                @pl.when(gi >= 1)
                def _():
                    wait_gather(2)
                    pltpu.async_copy(bf[2], acc.at[cx[2]], ssm[2], add=True)
            else:
                wait_gather(q)
                pltpu.async_copy(bf[q], acc.at[cx[q]], ssm[q], add=True)
        return carry

    lax.fori_loop(0, _NCHK // 3, group_fn, 0)
    wait_gather(2)
    pltpu.async_copy(bf[2], acc.at[cx[2]], ssm[2], add=True)
    for p in range(3):
        wait_scatter(p)
    plsc.subcore_barrier()

    last = _N - (_NSUB - 1) * _RPT

    @pl.when(s < _NSUB - 1)
    def _():
        pltpu.sync_copy(acc.at[pl.ds(s * _RPT, _RPT)],
                        out.at[c, pl.ds(s * _RPT, _RPT)])

    @pl.when(s == _NSUB - 1)
    def _():
        pltpu.sync_copy(acc.at[pl.ds((_NSUB - 1) * _RPT, last)],
                        out.at[c, pl.ds((_NSUB - 1) * _RPT, last)])


def _elu1(z):
    return 1.0 + jnp.where(z > 0, z, jnp.exp(z) - 1.0)


def _prep_body(x, wi, bi, wq, bq, wk, bk, wv, bv, rr, tt, d0, d1, tmp,
               z_o, q_o, h0_o, dis_o):
    h = jnp.maximum(jnp.dot(x[...], wi[...], preferred_element_type=jnp.float32)
                    + bi[...], 0.0)
    q = _elu1(jnp.dot(h, wq[...], preferred_element_type=jnp.float32) + bq[...])
    kf = _elu1(jnp.dot(h, wk[...], preferred_element_type=jnp.float32) + bk[...])
    v = jnp.dot(h, wv[...], preferred_element_type=jnp.float32) + bv[...]
    deg = d0[...] + d1[...] + 1.0
    dis = lax.rsqrt(deg)
    zm = (jnp.dot(kf, rr[...], preferred_element_type=jnp.float32)
          * jnp.dot(v, tt[...], preferred_element_type=jnp.float32))
    z_o[...] = jnp.concatenate([zm, kf], axis=1) * dis
    q_o[...] = q
    h0_o[...] = v * tmp[0, 0]
    dis_o[...] = dis


def _prep_call(x, wi, bi, wq, bq, wk, bk, wv, bv, rr, tt, d0, d1, tmp):
    full = lambda shape: pl.BlockSpec(shape, lambda i: (0, 0))
    rowb = lambda w: pl.BlockSpec((_BN, w), lambda i: (i, 0))
    return pl.pallas_call(
        _prep_body,
        grid=(_GRID,),
        in_specs=[
            rowb(_FIN), full((_FIN, _HID)), full((1, _HID)),
            full((_HID, _HID)), full((1, _HID)),
            full((_HID, _HID)), full((1, _HID)),
            full((_HID, _NCH)), full((1, _NCH)),
            full((_HID, _HID * _NCH)), full((_NCH, _HID * _NCH)),
            rowb(1), rowb(1), full((1, 3)),
        ],
        out_specs=[rowb(2 * _HALF), rowb(_HID), rowb(_NCH), rowb(1)],
        out_shape=[
            jax.ShapeDtypeStruct((_N, 2 * _HALF), jnp.float32),
            jax.ShapeDtypeStruct((_N, _HID), jnp.float32),
            jax.ShapeDtypeStruct((_N, _NCH), jnp.float32),
            jax.ShapeDtypeStruct((_N, 1), jnp.float32),
        ],
    )(x, wi, bi, wq, bq, wk, bk, wv, bv, rr, tt, d0, d1, tmp)


def _post_body(ya, yb, q, dis, hid, rr, ttt, tmp, ho_o, z_o):
    y = jnp.concatenate([ya[0], yb[0]], axis=1)
    d = dis[...]
    m = y[:, : _HID * _NCH] * d
    kf = y[:, _HID * _NCH:] * d
    qb = q[...]
    hh = jnp.dot(jnp.dot(qb, rr[...], preferred_element_type=jnp.float32) * m,
                 ttt[...], preferred_element_type=jnp.float32)
    cd = jnp.sum(qb * kf, axis=1, keepdims=True) + _CST
    ho_o[...] = hid[...] + tmp[0, 0] * (hh / cd)
    z_o[...] = y * (d * d)


def _post_call(y, q, dis, hid, rr, ttt, tmp):
    full = lambda shape: pl.BlockSpec(shape, lambda i: (0, 0))
    rowb = lambda w: pl.BlockSpec((_BN, w), lambda i: (i, 0))
    ya_spec = pl.BlockSpec((1, _BN, _HALF), lambda i: (0, i, 0))
    yb_spec = pl.BlockSpec((1, _BN, _HALF), lambda i: (1, i, 0))
    return pl.pallas_call(
        _post_body,
        grid=(_GRID,),
        in_specs=[
            ya_spec, yb_spec, rowb(_HID), rowb(1), rowb(_NCH),
            full((_HID, _HID * _NCH)), full((_HID * _NCH, _NCH)), full((1, 1)),
        ],
        out_specs=[rowb(_NCH), rowb(2 * _HALF)],
        out_shape=[
            jax.ShapeDtypeStruct((_N, _NCH), jnp.float32),
            jax.ShapeDtypeStruct((_N, 2 * _HALF), jnp.float32),
        ],
    )(y, y, q, dis, hid, rr, ttt, tmp)


_R_EXPAND = np.kron(np.eye(_HID), np.ones((1, _NCH))).astype(np.float32)
_T_EXPAND = np.kron(np.ones((1, _HID)), np.eye(_NCH)).astype(np.float32)


def kernel(x, edge_index, W_in, b_in, WQ, bQ, WK, bK, WV, bV, temp):
    row = edge_index[0]
    col = edge_index[1]
    npad = _EARR - _E
    rowp = jnp.concatenate([row, jnp.zeros((npad,), jnp.int32)])
    colp = jnp.concatenate([col, jnp.full((npad,), _N, jnp.int32)])

    degp = _get_deg_kernel()(colp)
    d0 = degp[0, :_N, None]
    d1 = degp[1, :_N, None]

    tmp = temp.reshape(1, 3)
    z0, q, hid0, dis = _prep_call(
        x, W_in.T, b_in.reshape(1, _HID), WQ.T, bQ.reshape(1, _HID),
        WK.T, bK.reshape(1, _HID), WV.T, bV.reshape(1, _NCH),
        _R_EXPAND, _T_EXPAND, d0, d1, tmp)

    ttt = np.ascontiguousarray(_T_EXPAND.T)
    t_steps = temp[1:].reshape(2, 1, 1)

    def body(carry, t_h):
        hid, z = carry
        y = _get_hop_kernel()(z.reshape(2 * _N, _HALF), rowp, colp)
        hid_n, z_n = _post_call(y, q, dis, hid, _R_EXPAND, ttt, t_h)
        return (hid_n, z_n), None

    (hid_f, _), _ = lax.scan(body, (hid0, z0), t_steps)
    return hid_f
